# baseline (speedup 1.0000x reference)
import functools
import math

import jax
import jax.numpy as jnp
from jax import lax
from jax.experimental import pallas as pl
from jax.experimental.pallas import tpu as pltpu
from jax.experimental.pallas import tpu_sc as plsc

D_MODEL = 64
SCALE = math.sqrt(D_MODEL)

_NC = 2
_NS = 16
_NW = _NC * _NS
_CHUNK = 128
_LANES = 16


@functools.lru_cache(maxsize=None)
def _make_sc_kernel(B):
    assert B % (_NW * _CHUNK) == 0
    rows_per_w = B // _NW
    nch = rows_per_w // _CHUNK

    mesh = plsc.VectorSubcoreMesh(core_axis_name="c", subcore_axis_name="s")

    @functools.partial(
        pl.kernel,
        mesh=mesh,
        compiler_params=pltpu.CompilerParams(use_tc_tiling_on_sc=False),
        out_type=jax.ShapeDtypeStruct((B, D_MODEL), jnp.float32),
        scratch_types=[
            pltpu.VMEM((nch, _CHUNK), jnp.int32),
            pltpu.VMEM((_CHUNK, D_MODEL), jnp.float32),
            pltpu.VMEM((_CHUNK, D_MODEL), jnp.float32),
            pltpu.SemaphoreType.DMA,
        ],
    )
    def k(x_hbm, table_hbm, out_hbm, idx_v, in_v, out_v, sem):
        wid = lax.axis_index("s") * _NC + lax.axis_index("c")
        base_idx_row = wid * nch
        base_out = wid * rows_per_w
        pltpu.sync_copy(x_hbm.at[pl.ds(base_idx_row, nch)], idx_v)

        def chunk_body(j, carry):
            pltpu.async_copy(table_hbm.at[idx_v.at[j]], in_v, sem).wait()

            def row_body(i, c):
                for kk in range(D_MODEL // _LANES):
                    sl = pl.ds(kk * _LANES, _LANES)
                    out_v[i, sl] = in_v[i, sl] * SCALE
                return c

            lax.fori_loop(0, _CHUNK, row_body, 0)
            pltpu.sync_copy(
                out_v, out_hbm.at[pl.ds(base_out + j * _CHUNK, _CHUNK)]
            )
            return carry

        lax.fori_loop(0, nch, chunk_body, 0)

    return k


def kernel(x, table):
    B = x.size
    x2 = x.reshape(-1, _CHUNK).astype(jnp.int32)
    out = _make_sc_kernel(B)(x2, table)
    return out.reshape(x.shape + (D_MODEL,))
